# baseline (speedup 1.0000x reference)
import functools

import jax
import jax.numpy as jnp
from jax import lax
from jax.experimental import pallas as pl
from jax.experimental.pallas import tpu as pltpu
from jax.experimental.pallas import tpu_sc as plsc

D_MODEL = 128
HALF = 64
SEQ_LEN = 512
N_PERSONS = 128
SEQ_TILE = 128
LANES = 16
N_WORKERS = 32

T_ROWS_PER_W = SEQ_LEN // N_WORKERS
P_WORKERS = N_PERSONS // LANES


def _renorm16_inplace(buf):
    row_idx = lax.iota(jnp.int32, LANES)

    @pl.loop(0, HALF, init_carry=jnp.zeros((LANES,), jnp.float32), unroll=4)
    def ss(c, acc):
        col = jnp.full((LANES,), c, jnp.int32)
        v = plsc.load_gather(buf, [row_idx, col])
        return acc + v * v

    ss = jnp.maximum(ss, 1e-30)
    i = plsc.bitcast(ss, jnp.int32)
    i = 0x5F3759DF - (i >> 1)
    y = plsc.bitcast(i, jnp.float32)
    for _ in range(4):
        y = y * (1.5 - 0.5 * ss * y * y)
    norm = ss * y
    scale = jnp.where(norm > 1.0, 1.0 / (norm + 1e-7), 1.0)

    @pl.loop(0, HALF, unroll=4)
    def _(c):
        col = jnp.full((LANES,), c, jnp.int32)
        v = plsc.load_gather(buf, [row_idx, col])
        plsc.store_scatter(buf, [row_idx, col], v * scale)


@functools.partial(
    pl.kernel,
    mesh=plsc.VectorSubcoreMesh(core_axis_name="c", subcore_axis_name="s"),
    out_type=jax.ShapeDtypeStruct((SEQ_LEN + N_PERSONS, HALF), jnp.float32),
    scratch_types=[
        pltpu.VMEM((T_ROWS_PER_W, HALF), jnp.float32),
        pltpu.VMEM((LANES, HALF), jnp.float32),
    ],
    compiler_params=pltpu.CompilerParams(needs_layout_passes=False),
)
def _sc_encode(time_hbm, person_hbm, enc_out, t_buf, p_buf):
    wid = lax.axis_index("s") * 2 + lax.axis_index("c")
    tb = wid * T_ROWS_PER_W
    pltpu.sync_copy(time_hbm.at[pl.ds(tb, T_ROWS_PER_W)], t_buf)
    _renorm16_inplace(t_buf)
    pltpu.sync_copy(t_buf, enc_out.at[pl.ds(tb, T_ROWS_PER_W)])

    @pl.when(wid < P_WORKERS)
    def _person():
        pb = wid * LANES
        pltpu.sync_copy(person_hbm.at[pl.ds(pb, LANES)], p_buf)
        _renorm16_inplace(p_buf)
        pltpu.sync_copy(p_buf, enc_out.at[pl.ds(SEQ_LEN + pb, LANES)])


def _interleave_add(t, p, x_block):
    rows = lax.broadcasted_iota(jnp.int32, (HALF, D_MODEL), 0)
    cols = lax.broadcasted_iota(jnp.int32, (HALF, D_MODEL), 1)
    e_even = (cols == 2 * rows).astype(jnp.float32)
    e_odd = (cols == 2 * rows + 1).astype(jnp.float32)
    t_full = jnp.dot(t, e_even, preferred_element_type=jnp.float32)
    p_full = jnp.dot(p, e_odd, preferred_element_type=jnp.float32)
    return x_block + t_full[None, :, None, :] + p_full[None, None, :, :]


def _tc_body_enc(t_ref, p_ref, x_ref, out_ref):
    out_ref[...] = _interleave_add(t_ref[...], p_ref[...], x_ref[...])


def _tc_renorm(rows, max_norm=1.0):
    norm = jnp.sqrt(jnp.sum(rows * rows, axis=-1, keepdims=True))
    scale = jnp.where(norm > max_norm, max_norm / (norm + 1e-7), 1.0)
    return rows * scale


def _tc_body_tables(t_ref, p_ref, x_ref, out_ref):
    out_ref[...] = _interleave_add(
        _tc_renorm(t_ref[...]), _tc_renorm(p_ref[...]), x_ref[...]
    )


@jax.jit
def kernel(x, time_table, person_table):
    B, S, N, D = x.shape
    n_s = S // SEQ_TILE
    x_spec4 = pl.BlockSpec((1, SEQ_TILE, N, D), lambda s, b: (b, s, 0, 0))

    enc = _sc_encode(time_table, person_table)

    out0 = pl.pallas_call(
        _tc_body_tables,
        grid=(n_s, 1),
        in_specs=[
            pl.BlockSpec((SEQ_TILE, HALF), lambda s, b: (s, 0)),
            pl.BlockSpec((N, HALF), lambda s, b: (0, 0)),
            x_spec4,
        ],
        out_specs=x_spec4,
        out_shape=jax.ShapeDtypeStruct((1, S, N, D), x.dtype),
    )(time_table, person_table, x)

    out_rest = pl.pallas_call(
        _tc_body_enc,
        grid=(n_s, B - 1),
        in_specs=[
            pl.BlockSpec((SEQ_TILE, HALF), lambda s, b: (s, 0)),
            pl.BlockSpec((N, HALF), lambda s, b: (S // N, 0)),
            pl.BlockSpec((1, SEQ_TILE, N, D), lambda s, b: (b + 1, s, 0, 0)),
        ],
        out_specs=x_spec4,
        out_shape=jax.ShapeDtypeStruct((B - 1, S, N, D), x.dtype),
    )(enc, enc, x)

    return lax.concatenate([out0, out_rest], 0)


# speedup vs baseline: 1.7907x; 1.7907x over previous
import functools

import jax
import jax.numpy as jnp
from jax import lax
from jax.experimental import pallas as pl
from jax.experimental.pallas import tpu as pltpu
from jax.experimental.pallas import tpu_sc as plsc

D_MODEL = 128
HALF = 64
SEQ_LEN = 512
N_PERSONS = 128
SEQ_TILE = 128
LANES = 16
N_WORKERS = 32

T_ROWS_PER_W = SEQ_LEN // N_WORKERS
P_WORKERS = N_PERSONS // LANES


def _renorm16_inplace(buf):
    row_idx = lax.iota(jnp.int32, LANES)

    @pl.loop(0, HALF, init_carry=jnp.zeros((LANES,), jnp.float32), unroll=4)
    def ss(c, acc):
        col = jnp.full((LANES,), c, jnp.int32)
        v = plsc.load_gather(buf, [row_idx, col])
        return acc + v * v

    ss = jnp.maximum(ss, 1e-30)
    i = plsc.bitcast(ss, jnp.int32)
    i = 0x5F3759DF - (i >> 1)
    y = plsc.bitcast(i, jnp.float32)
    for _ in range(4):
        y = y * (1.5 - 0.5 * ss * y * y)
    norm = ss * y
    scale = jnp.where(norm > 1.0, 1.0 / (norm + 1e-7), 1.0)

    @pl.loop(0, HALF, unroll=4)
    def _(c):
        col = jnp.full((LANES,), c, jnp.int32)
        v = plsc.load_gather(buf, [row_idx, col])
        plsc.store_scatter(buf, [row_idx, col], v * scale)


@functools.partial(
    pl.kernel,
    mesh=plsc.VectorSubcoreMesh(core_axis_name="c", subcore_axis_name="s"),
    out_type=jax.ShapeDtypeStruct((SEQ_LEN + N_PERSONS, HALF), jnp.float32),
    scratch_types=[
        pltpu.VMEM((T_ROWS_PER_W, HALF), jnp.float32),
        pltpu.VMEM((LANES, HALF), jnp.float32),
    ],
    compiler_params=pltpu.CompilerParams(needs_layout_passes=False),
)
def _sc_encode(time_hbm, person_hbm, enc_out, t_buf, p_buf):
    wid = lax.axis_index("s") * 2 + lax.axis_index("c")
    tb = wid * T_ROWS_PER_W
    pltpu.sync_copy(time_hbm.at[pl.ds(tb, T_ROWS_PER_W)], t_buf)
    _renorm16_inplace(t_buf)
    pltpu.sync_copy(t_buf, enc_out.at[pl.ds(tb, T_ROWS_PER_W)])

    @pl.when(wid < P_WORKERS)
    def _person():
        pb = wid * LANES
        pltpu.sync_copy(person_hbm.at[pl.ds(pb, LANES)], p_buf)
        _renorm16_inplace(p_buf)
        pltpu.sync_copy(p_buf, enc_out.at[pl.ds(SEQ_LEN + pb, LANES)])


def _interleave_add(t, p, x_block):
    rows = lax.broadcasted_iota(jnp.int32, (HALF, D_MODEL), 0)
    cols = lax.broadcasted_iota(jnp.int32, (HALF, D_MODEL), 1)
    e_even = (cols == 2 * rows).astype(jnp.float32)
    e_odd = (cols == 2 * rows + 1).astype(jnp.float32)
    t_full = jnp.dot(t, e_even, preferred_element_type=jnp.float32)
    p_full = jnp.dot(p, e_odd, preferred_element_type=jnp.float32)
    return x_block + t_full[None, :, None, :] + p_full[None, None, :, :]


def _tc_body_enc(t_ref, p_ref, x_ref, alias_ref, out_ref):
    del alias_ref
    out_ref[...] = _interleave_add(t_ref[...], p_ref[...], x_ref[...])


def _tc_renorm(rows, max_norm=1.0):
    norm = jnp.sqrt(jnp.sum(rows * rows, axis=-1, keepdims=True))
    scale = jnp.where(norm > max_norm, max_norm / (norm + 1e-7), 1.0)
    return rows * scale


def _tc_body_tables(t_ref, p_ref, x_ref, out_ref):
    out_ref[...] = _interleave_add(
        _tc_renorm(t_ref[...]), _tc_renorm(p_ref[...]), x_ref[...]
    )


@jax.jit
def kernel(x, time_table, person_table):
    B, S, N, D = x.shape
    n_s = S // SEQ_TILE
    x_spec4 = pl.BlockSpec((1, SEQ_TILE, N, D), lambda s, b: (b, s, 0, 0))

    enc = _sc_encode(time_table, person_table)

    out0 = pl.pallas_call(
        _tc_body_tables,
        grid=(n_s, 1),
        in_specs=[
            pl.BlockSpec((SEQ_TILE, HALF), lambda s, b: (s, 0)),
            pl.BlockSpec((N, HALF), lambda s, b: (0, 0)),
            x_spec4,
        ],
        out_specs=x_spec4,
        out_shape=jax.ShapeDtypeStruct(x.shape, x.dtype),
    )(time_table, person_table, x)

    return pl.pallas_call(
        _tc_body_enc,
        grid=(n_s, B - 1),
        in_specs=[
            pl.BlockSpec((SEQ_TILE, HALF), lambda s, b: (s, 0)),
            pl.BlockSpec((N, HALF), lambda s, b: (S // N, 0)),
            pl.BlockSpec((1, SEQ_TILE, N, D), lambda s, b: (b + 1, s, 0, 0)),
            pl.BlockSpec(memory_space=pl.ANY),
        ],
        out_specs=pl.BlockSpec((1, SEQ_TILE, N, D), lambda s, b: (b + 1, s, 0, 0)),
        out_shape=jax.ShapeDtypeStruct(x.shape, x.dtype),
        input_output_aliases={3: 0},
    )(enc, enc, x, out0)


# speedup vs baseline: 2.0734x; 1.1579x over previous
import functools

import jax
import jax.numpy as jnp
from jax import lax
from jax.experimental import pallas as pl
from jax.experimental.pallas import tpu as pltpu
from jax.experimental.pallas import tpu_sc as plsc

D_MODEL = 128
HALF = 64
SEQ_LEN = 512
N_PERSONS = 128
SEQ_TILE = 128
LANES = 16
N_WORKERS = 32

T_ROWS_PER_W = SEQ_LEN // N_WORKERS
P_WORKERS = N_PERSONS // LANES


def _renorm16_inplace(buf):
    row_idx = lax.iota(jnp.int32, LANES)

    @pl.loop(0, HALF, init_carry=jnp.zeros((LANES,), jnp.float32), unroll=4)
    def ss(c, acc):
        col = jnp.full((LANES,), c, jnp.int32)
        v = plsc.load_gather(buf, [row_idx, col])
        return acc + v * v

    ss = jnp.maximum(ss, 1e-30)
    i = plsc.bitcast(ss, jnp.int32)
    i = 0x5F3759DF - (i >> 1)
    y = plsc.bitcast(i, jnp.float32)
    for _ in range(4):
        y = y * (1.5 - 0.5 * ss * y * y)
    norm = ss * y
    scale = jnp.where(norm > 1.0, 1.0 / (norm + 1e-7), 1.0)

    @pl.loop(0, HALF, unroll=4)
    def _(c):
        col = jnp.full((LANES,), c, jnp.int32)
        v = plsc.load_gather(buf, [row_idx, col])
        plsc.store_scatter(buf, [row_idx, col], v * scale)


@functools.partial(
    pl.kernel,
    mesh=plsc.VectorSubcoreMesh(core_axis_name="c", subcore_axis_name="s"),
    out_type=jax.ShapeDtypeStruct((SEQ_LEN + N_PERSONS, HALF), jnp.float32),
    scratch_types=[
        pltpu.VMEM((T_ROWS_PER_W, HALF), jnp.float32),
        pltpu.VMEM((LANES, HALF), jnp.float32),
    ],
    compiler_params=pltpu.CompilerParams(needs_layout_passes=False),
)
def _sc_encode(time_hbm, person_hbm, enc_out, t_buf, p_buf):
    wid = lax.axis_index("s") * 2 + lax.axis_index("c")
    tb = wid * T_ROWS_PER_W
    pltpu.sync_copy(time_hbm.at[pl.ds(tb, T_ROWS_PER_W)], t_buf)
    _renorm16_inplace(t_buf)
    pltpu.sync_copy(t_buf, enc_out.at[pl.ds(tb, T_ROWS_PER_W)])

    @pl.when(wid < P_WORKERS)
    def _person():
        pb = wid * LANES
        pltpu.sync_copy(person_hbm.at[pl.ds(pb, LANES)], p_buf)
        _renorm16_inplace(p_buf)
        pltpu.sync_copy(p_buf, enc_out.at[pl.ds(SEQ_LEN + pb, LANES)])


def _interleave_add(t, p, x_block):
    rows = lax.broadcasted_iota(jnp.int32, (HALF, D_MODEL), 0)
    cols = lax.broadcasted_iota(jnp.int32, (HALF, D_MODEL), 1)
    e_even = (cols == 2 * rows).astype(jnp.float32)
    e_odd = (cols == 2 * rows + 1).astype(jnp.float32)
    t_full = jnp.dot(t, e_even, preferred_element_type=jnp.float32)
    p_full = jnp.dot(p, e_odd, preferred_element_type=jnp.float32)
    return x_block + t_full[None, :, None, :] + p_full[None, None, :, :]


def _tc_body_enc(t_ref, p_ref, x_ref, alias_ref, out_ref):
    del alias_ref
    out_ref[...] = _interleave_add(t_ref[...], p_ref[...], x_ref[...])


def _tc_renorm(rows, max_norm=1.0):
    norm = jnp.sqrt(jnp.sum(rows * rows, axis=-1, keepdims=True))
    scale = jnp.where(norm > max_norm, max_norm / (norm + 1e-7), 1.0)
    return rows * scale


def _tc_body_tables(t_ref, p_ref, x_ref, out_ref):
    out_ref[...] = _interleave_add(
        _tc_renorm(t_ref[...]), _tc_renorm(p_ref[...]), x_ref[...]
    )


def _tc_body_tables2(t_ref, p_ref, x_ref, alias_ref, out_ref):
    del alias_ref
    out_ref[...] = _interleave_add(
        _tc_renorm(t_ref[...]), _tc_renorm(p_ref[...]), x_ref[...]
    )


@jax.jit
def kernel(x, time_table, person_table):
    B, S, N, D = x.shape
    n_s = S // SEQ_TILE
    x_spec4 = pl.BlockSpec((1, SEQ_TILE, N, D), lambda s, b: (b, s, 0, 0))


    out0 = pl.pallas_call(
        _tc_body_tables,
        grid=(n_s, 1),
        in_specs=[
            pl.BlockSpec((SEQ_TILE, HALF), lambda s, b: (s, 0)),
            pl.BlockSpec((N, HALF), lambda s, b: (0, 0)),
            x_spec4,
        ],
        out_specs=x_spec4,
        out_shape=jax.ShapeDtypeStruct(x.shape, x.dtype),
    )(time_table, person_table, x)

    return pl.pallas_call(
        _tc_body_tables2,
        grid=(n_s, B - 1),
        in_specs=[
            pl.BlockSpec((SEQ_TILE, HALF), lambda s, b: (s, 0)),
            pl.BlockSpec((N, HALF), lambda s, b: (0, 0)),
            pl.BlockSpec((1, SEQ_TILE, N, D), lambda s, b: (b + 1, s, 0, 0)),
            pl.BlockSpec(memory_space=pl.ANY),
        ],
        out_specs=pl.BlockSpec((1, SEQ_TILE, N, D), lambda s, b: (b + 1, s, 0, 0)),
        out_shape=jax.ShapeDtypeStruct(x.shape, x.dtype),
        input_output_aliases={3: 0},
    )(time_table, person_table, x, out0)
